# baseline (speedup 1.0000x reference)
import functools
import math

import jax
import jax.numpy as jnp
from jax import lax
from jax.experimental import pallas as pl
from jax.experimental.pallas import tpu as pltpu
from jax.experimental.pallas import tpu_sc as plsc

N = 10000
NT = N + 16
E = 320000
EH = 161280
EP2 = 2 * EH
DIM = 128
NC, NS, L = 2, 16, 16
NW = NC * NS
EPW = EH // NW
C = 80
NCHUNK = EPW // C
NPAD = 10240
ACC_LEN = NPAD * 4
INV_SQRT_D = 1.0 / math.sqrt(DIM)


def _node_proj_body(x_ref, wq_ref, bq_ref, wk_ref, bk_ref, wv_ref, bv_ref,
                    wf_ref, bf_ref, q_ref, k_ref, g_ref):
    x = x_ref[...]
    q_ref[...] = x @ wq_ref[...] + bq_ref[...]
    k_ref[...] = x @ wk_ref[...] + bk_ref[...]
    w_g = wv_ref[...] @ wf_ref[...]
    c_g = bv_ref[...] @ wf_ref[...] + bf_ref[...]
    g_ref[...] = x @ w_g + c_g


def _node_proj(x, W_Q, b_Q, W_K, b_K, W_V, b_V, wF_W, wF_b):
    return pl.pallas_call(
        _node_proj_body,
        out_shape=(
            jax.ShapeDtypeStruct((NT, DIM), jnp.float32),
            jax.ShapeDtypeStruct((NT, DIM), jnp.float32),
            jax.ShapeDtypeStruct((NT, 1), jnp.float32),
        ),
    )(x, W_Q, b_Q.reshape(1, DIM), W_K, b_K.reshape(1, DIM),
      W_V, b_V.reshape(1, DIM), wF_W, wF_b.reshape(1, 1))


_EB = 7680


def _edge_mlp_body(ev_ref, w0_ref, b0_ref, w1_ref, b1_ref, w2_ref, b2_ref,
                   bias_ref):
    ev = ev_ref[...]
    ln = jnp.sqrt(jnp.sum(ev * ev, axis=0, keepdims=True))
    attr = jnp.concatenate([ev[0:3], ln], axis=0)
    h = w0_ref[...] @ attr + b0_ref[...]
    h = h * jax.nn.sigmoid(h)
    h = w1_ref[...] @ h + b1_ref[...]
    h = h * jax.nn.sigmoid(h)
    bias_ref[...] = (w2_ref[...] @ h + b2_ref[...]).reshape(1, 1, _EB)


def _edge_mlp(evT, mW0, mb0, mW1, mb1, mW2, mb2, blk_off):
    nblk = EH // _EB
    w_spec = lambda shp: pl.BlockSpec(shp, lambda e: (0, 0))
    return pl.pallas_call(
        _edge_mlp_body,
        grid=(nblk,),
        in_specs=[
            pl.BlockSpec((8, _EB), lambda e: (0, e + blk_off)),
            w_spec((DIM, 4)), w_spec((DIM, 1)),
            w_spec((DIM, DIM)), w_spec((DIM, 1)),
            w_spec((1, DIM)), w_spec((1, 1)),
        ],
        out_specs=pl.BlockSpec((1, 1, _EB), lambda e: (e, 0, 0)),
        out_shape=jax.ShapeDtypeStruct((EH // _EB, 1, _EB), jnp.float32),
    )(evT, mW0.T, mb0.reshape(DIM, 1), mW1.T, mb1.reshape(DIM, 1),
      mW2.T, mb2.reshape(1, 1))


def _splat(v):
    return jnp.full((L,), v, jnp.int32)


def _edge_pass_body(hoff, q_hbm, k_hbm, g_hbm, i_hbm, j_hbm, ev_hbm,
                    bias_hbm, out_hbm, i_v, j_v, g_v, qbuf0, kbuf0, evbuf0,
                    qbuf1, kbuf1, evbuf1, acc, sem_q, sem_k, sem_e):
    wid = lax.axis_index("s") * NC + lax.axis_index("c")
    base = hoff + wid * EPW
    bbase = wid * EPW

    zero = jnp.zeros((L,), jnp.float32)

    def zbody(t, carry):
        acc[pl.ds(t * L, L)] = zero
        return carry

    lax.fori_loop(0, ACC_LEN // L, zbody, 0, unroll=8)

    pltpu.sync_copy(i_hbm.at[pl.ds(base, EPW)], i_v)
    pltpu.sync_copy(j_hbm.at[pl.ds(base, EPW)], j_v)
    pltpu.sync_copy(g_hbm, g_v)

    e16 = lax.iota(jnp.int32, L)

    def copies(c, qb, kb, eb):
        off = c * C
        cps = [
            pltpu.make_async_copy(q_hbm.at[j_v.at[pl.ds(off, C)]], qb, sem_q),
            pltpu.make_async_copy(k_hbm.at[i_v.at[pl.ds(off, C)]], kb, sem_k),
        ]
        for p in range(3):
            cps.append(pltpu.make_async_copy(
                ev_hbm.at[pl.ds(p * EP2 + base + off, C)],
                eb.at[pl.ds(p * C, C)], sem_e))
        cps.append(pltpu.make_async_copy(
            bias_hbm.at[pl.ds(bbase + off, C)],
            eb.at[pl.ds(3 * C, C)], sem_e))
        return cps

    def issue(c, qb, kb, eb):
        for cp in copies(c, qb, kb, eb):
            cp.start()

    def compute(c, qb, kb, eb):
        off = c * C
        for cp in copies(c, qb, kb, eb):
            cp.wait()
        for g in range(C // L):
            eg = g * L + e16

            def ebody(le, s16):
                e = g * L + le
                p = qb[e, pl.ds(0, L)] * kb[e, pl.ds(0, L)]
                for cc in range(1, DIM // L):
                    p = p + qb[e, pl.ds(cc * L, L)] * kb[e, pl.ds(cc * L, L)]
                tot = jnp.full((L,), plsc.cumsum(p)[L - 1])
                return jnp.where(e16 == le, tot, s16)

            s = lax.fori_loop(0, L, ebody, jnp.zeros((L,), jnp.float32),
                              unroll=4)
            s = s * INV_SQRT_D + eb[pl.ds(3 * C + g * L, L)]
            w = jnp.exp(s)
            iv = i_v[pl.ds(off + g * L, L)]
            gate = plsc.load_gather(g_v, [iv])
            ev0 = eb[pl.ds(g * L, L)]
            ev1 = eb[pl.ds(C + g * L, L)]
            ev2 = eb[pl.ds(2 * C + g * L, L)]
            wg = w * gate
            jv = j_v[pl.ds(off + g * L, L)]
            plsc.addupdate_scatter(acc, [jv], w)
            plsc.addupdate_scatter(acc, [jv + NPAD], wg * ev0)
            plsc.addupdate_scatter(acc, [jv + 2 * NPAD], wg * ev1)
            plsc.addupdate_scatter(acc, [jv + 3 * NPAD], wg * ev2)

    issue(0, qbuf0, kbuf0, evbuf0)

    def pair_body(t, carry):
        c = 2 * t
        issue(c + 1, qbuf1, kbuf1, evbuf1)
        compute(c, qbuf0, kbuf0, evbuf0)
        issue(c + 2, qbuf0, kbuf0, evbuf0)
        compute(c + 1, qbuf1, kbuf1, evbuf1)
        return carry

    lax.fori_loop(0, (NCHUNK - 1) // 2, pair_body, 0)
    compute(NCHUNK - 1, qbuf0, kbuf0, evbuf0)

    pltpu.sync_copy(acc, out_hbm.at[wid])


def _edge_pass(hoff, q_all, k_all, g_all, i_arr, j_arr, ev3_flat, bias_flat):
    mesh = plsc.VectorSubcoreMesh(core_axis_name="c", subcore_axis_name="s",
                                  num_cores=NC, num_subcores=NS)
    f = pl.kernel(
        functools.partial(_edge_pass_body, hoff),
        out_type=jax.ShapeDtypeStruct((NW, ACC_LEN), jnp.float32),
        mesh=mesh,
        scratch_types=[
            pltpu.VMEM((EPW,), jnp.int32),
            pltpu.VMEM((EPW,), jnp.int32),
            pltpu.VMEM((NT,), jnp.float32),
            pltpu.VMEM((C, DIM), jnp.float32),
            pltpu.VMEM((C, DIM), jnp.float32),
            pltpu.VMEM((C * 4,), jnp.float32),
            pltpu.VMEM((C, DIM), jnp.float32),
            pltpu.VMEM((C, DIM), jnp.float32),
            pltpu.VMEM((C * 4,), jnp.float32),
            pltpu.VMEM((ACC_LEN,), jnp.float32),
            pltpu.SemaphoreType.DMA,
            pltpu.SemaphoreType.DMA,
            pltpu.SemaphoreType.DMA,
        ],
        compiler_params=pltpu.CompilerParams(needs_layout_passes=False),
    )
    return f(q_all, k_all, g_all, i_arr, j_arr, ev3_flat, bias_flat)


def _finalize_body(acc_ref, out_ref):
    a = acc_ref[...].reshape(2 * NW, 4, NPAD)
    a = jnp.sum(a, axis=0)
    out_ref[...] = a[1:4] / (a[0:1] + 1e-16)


def _finalize(acc_all):
    return pl.pallas_call(
        _finalize_body,
        out_shape=jax.ShapeDtypeStruct((3, NPAD), jnp.float32),
    )(acc_all)


def kernel(x, edge_index, edge_vec, W_Q, b_Q, W_K, b_K, W_V, b_V,
           mW0, mb0, mW1, mb1, mW2, mb2, wF_W, wF_b):
    x_pad = jnp.concatenate([x, jnp.zeros((NT - N, DIM), jnp.float32)])
    q_all, k_all, g_all = _node_proj(x_pad, W_Q, b_Q, W_K, b_K, W_V, b_V,
                                     wF_W, wF_b)
    g_flat = g_all.reshape(NT)
    pad_e = EP2 - E
    i_arr = jnp.concatenate([edge_index[0],
                             jnp.full((pad_e,), N, jnp.int32)])
    j_arr = jnp.concatenate([edge_index[1],
                             jnp.full((pad_e,), N, jnp.int32)])
    evP = jnp.pad(edge_vec.T, ((0, 0), (0, pad_e)))
    evP8 = jnp.concatenate([evP, jnp.zeros((5, EP2), jnp.float32)], axis=0)
    ev3 = evP.reshape(3 * EP2)
    nb = EH // _EB
    bias1 = _edge_mlp(evP8, mW0, mb0, mW1, mb1, mW2, mb2, 0).reshape(EH)
    bias2 = _edge_mlp(evP8, mW0, mb0, mW1, mb1, mW2, mb2, nb).reshape(EH)
    acc1 = _edge_pass(0, q_all, k_all, g_flat, i_arr, j_arr, ev3, bias1)
    acc2 = _edge_pass(EH, q_all, k_all, g_flat, i_arr, j_arr, ev3, bias2)
    acc = jnp.concatenate([acc1, acc2]).reshape(2 * NW * 4, NPAD)
    out = _finalize(acc)
    return out[:, :N].T


# speedup vs baseline: 1.1849x; 1.1849x over previous
import functools
import math

import jax
import jax.numpy as jnp
from jax import lax
from jax.experimental import pallas as pl
from jax.experimental.pallas import tpu as pltpu
from jax.experimental.pallas import tpu_sc as plsc

N = 10000
E = 320000
DIM = 128
NC, NS, L = 2, 16, 16
NW = NC * NS
EPW = E // NW
C = 80
NCHUNK = EPW // C
NPAD = 10240
ACC_LEN = NPAD * 4
INV_SQRT_D = 1.0 / math.sqrt(DIM)


def _node_proj_body(x_ref, wq_ref, bq_ref, wk_ref, bk_ref, wv_ref, bv_ref,
                    wf_ref, bf_ref, q_ref, k_ref, g_ref):
    x = x_ref[...]
    q_ref[...] = x @ wq_ref[...] + bq_ref[...]
    k_ref[...] = x @ wk_ref[...] + bk_ref[...]
    w_g = wv_ref[...] @ wf_ref[...]
    c_g = bv_ref[...] @ wf_ref[...] + bf_ref[...]
    g_ref[...] = x @ w_g + c_g


def _node_proj(x, W_Q, b_Q, W_K, b_K, W_V, b_V, wF_W, wF_b):
    return pl.pallas_call(
        _node_proj_body,
        out_shape=(
            jax.ShapeDtypeStruct((N, DIM), jnp.float32),
            jax.ShapeDtypeStruct((N, DIM), jnp.float32),
            jax.ShapeDtypeStruct((N, 1), jnp.float32),
        ),
    )(x, W_Q, b_Q.reshape(1, DIM), W_K, b_K.reshape(1, DIM),
      W_V, b_V.reshape(1, DIM), wF_W, wF_b.reshape(1, 1))


_EB = 6400


def _edge_mlp_body(ev_ref, w0_ref, b0_ref, w1_ref, b1_ref, w2_ref, b2_ref,
                   bias_ref):
    ev = ev_ref[...]
    ln = jnp.sqrt(jnp.sum(ev * ev, axis=0, keepdims=True))
    attr = jnp.concatenate([ev, ln], axis=0)
    h = w0_ref[...] @ attr + b0_ref[...]
    h = h * jax.nn.sigmoid(h)
    h = w1_ref[...] @ h + b1_ref[...]
    h = h * jax.nn.sigmoid(h)
    bias_ref[...] = (w2_ref[...] @ h + b2_ref[...]).reshape(1, 1, _EB)


def _edge_mlp(evT, mW0, mb0, mW1, mb1, mW2, mb2):
    nblk = E // _EB
    w_spec = lambda shp: pl.BlockSpec(shp, lambda e: (0, 0))
    return pl.pallas_call(
        _edge_mlp_body,
        grid=(nblk,),
        in_specs=[
            pl.BlockSpec((3, _EB), lambda e: (0, e)),
            w_spec((DIM, 4)), w_spec((DIM, 1)),
            w_spec((DIM, DIM)), w_spec((DIM, 1)),
            w_spec((1, DIM)), w_spec((1, 1)),
        ],
        out_specs=pl.BlockSpec((1, 1, _EB), lambda e: (e, 0, 0)),
        out_shape=jax.ShapeDtypeStruct((E // _EB, 1, _EB), jnp.float32),
    )(evT, mW0.T, mb0.reshape(DIM, 1), mW1.T, mb1.reshape(DIM, 1),
      mW2.T, mb2.reshape(1, 1))


def _splat(v):
    return jnp.full((L,), v, jnp.int32)


def _edge_pass_body(q_hbm, k_hbm, g_hbm, i_hbm, j_hbm, ev_hbm, bias_hbm,
                    out_hbm, i_v, j_v, g_v, qbuf0, kbuf0, evbuf0, qbuf1,
                    kbuf1, evbuf1, acc, sem_q, sem_k, sem_e):
    wid = lax.axis_index("s") * NC + lax.axis_index("c")
    base = wid * EPW

    zero = jnp.zeros((L,), jnp.float32)

    def zbody(t, carry):
        acc[pl.ds(t * L, L)] = zero
        return carry

    lax.fori_loop(0, ACC_LEN // L, zbody, 0, unroll=8)

    pltpu.sync_copy(i_hbm.at[pl.ds(base, EPW)], i_v)
    pltpu.sync_copy(j_hbm.at[pl.ds(base, EPW)], j_v)
    pltpu.sync_copy(g_hbm, g_v)

    e16 = lax.iota(jnp.int32, L)

    def copies(c, qb, kb, eb):
        off = c * C
        cps = [
            pltpu.make_async_copy(q_hbm.at[j_v.at[pl.ds(off, C)]], qb, sem_q),
            pltpu.make_async_copy(k_hbm.at[i_v.at[pl.ds(off, C)]], kb, sem_k),
        ]
        for p in range(3):
            cps.append(pltpu.make_async_copy(
                ev_hbm.at[pl.ds(p * E + base + off, C)],
                eb.at[pl.ds(p * C, C)], sem_e))
        cps.append(pltpu.make_async_copy(
            bias_hbm.at[pl.ds(base + off, C)],
            eb.at[pl.ds(3 * C, C)], sem_e))
        return cps

    def issue(c, qb, kb, eb):
        for cp in copies(c, qb, kb, eb):
            cp.start()

    def compute(c, qb, kb, eb):
        off = c * C
        for cp in copies(c, qb, kb, eb):
            cp.wait()
        for g in range(C // L):
            eg = g * L + e16

            def ebody(le, s16):
                e = g * L + le
                p = qb[e, pl.ds(0, L)] * kb[e, pl.ds(0, L)]
                for cc in range(1, DIM // L):
                    p = p + qb[e, pl.ds(cc * L, L)] * kb[e, pl.ds(cc * L, L)]
                tot = jnp.full((L,), plsc.cumsum(p)[L - 1])
                return jnp.where(e16 == le, tot, s16)

            s = lax.fori_loop(0, L, ebody, jnp.zeros((L,), jnp.float32),
                              unroll=8)
            s = s * INV_SQRT_D + eb[pl.ds(3 * C + g * L, L)]
            w = jnp.exp(s)
            iv = i_v[pl.ds(off + g * L, L)]
            gate = plsc.load_gather(g_v, [iv])
            ev0 = eb[pl.ds(g * L, L)]
            ev1 = eb[pl.ds(C + g * L, L)]
            ev2 = eb[pl.ds(2 * C + g * L, L)]
            wg = w * gate
            jv = j_v[pl.ds(off + g * L, L)]
            plsc.addupdate_scatter(acc, [jv], w)
            plsc.addupdate_scatter(acc, [jv + NPAD], wg * ev0)
            plsc.addupdate_scatter(acc, [jv + 2 * NPAD], wg * ev1)
            plsc.addupdate_scatter(acc, [jv + 3 * NPAD], wg * ev2)

    issue(0, qbuf0, kbuf0, evbuf0)

    def pair_body(t, carry):
        c = 2 * t
        issue(c + 1, qbuf1, kbuf1, evbuf1)
        compute(c, qbuf0, kbuf0, evbuf0)
        issue(c + 2, qbuf0, kbuf0, evbuf0)
        compute(c + 1, qbuf1, kbuf1, evbuf1)
        return carry

    lax.fori_loop(0, (NCHUNK - 1) // 2, pair_body, 0)
    compute(NCHUNK - 1, qbuf0, kbuf0, evbuf0)

    pltpu.sync_copy(acc, out_hbm.at[wid])


def _edge_pass(q_all, k_all, g_all, i_arr, j_arr, ev3_flat, bias_flat):
    mesh = plsc.VectorSubcoreMesh(core_axis_name="c", subcore_axis_name="s",
                                  num_cores=NC, num_subcores=NS)
    f = pl.kernel(
        _edge_pass_body,
        out_type=jax.ShapeDtypeStruct((NW, ACC_LEN), jnp.float32),
        mesh=mesh,
        scratch_types=[
            pltpu.VMEM((EPW,), jnp.int32),
            pltpu.VMEM((EPW,), jnp.int32),
            pltpu.VMEM((N,), jnp.float32),
            pltpu.VMEM((C, DIM), jnp.float32),
            pltpu.VMEM((C, DIM), jnp.float32),
            pltpu.VMEM((C * 4,), jnp.float32),
            pltpu.VMEM((C, DIM), jnp.float32),
            pltpu.VMEM((C, DIM), jnp.float32),
            pltpu.VMEM((C * 4,), jnp.float32),
            pltpu.VMEM((ACC_LEN,), jnp.float32),
            pltpu.SemaphoreType.DMA,
            pltpu.SemaphoreType.DMA,
            pltpu.SemaphoreType.DMA,
        ],
        compiler_params=pltpu.CompilerParams(needs_layout_passes=False),
    )
    return f(q_all, k_all, g_all, i_arr, j_arr, ev3_flat, bias_flat)


def _finalize_body(acc_ref, out_ref):
    a = acc_ref[...].reshape(NW, 4, NPAD)
    a = jnp.sum(a, axis=0)
    out_ref[...] = a[1:4] / (a[0:1] + 1e-16)


def _finalize(acc_all):
    return pl.pallas_call(
        _finalize_body,
        out_shape=jax.ShapeDtypeStruct((3, NPAD), jnp.float32),
    )(acc_all)


def kernel(x, edge_index, edge_vec, W_Q, b_Q, W_K, b_K, W_V, b_V,
           mW0, mb0, mW1, mb1, mW2, mb2, wF_W, wF_b):
    q_all, k_all, g_all = _node_proj(x, W_Q, b_Q, W_K, b_K, W_V, b_V,
                                     wF_W, wF_b)
    evT = edge_vec.T
    bias = _edge_mlp(evT, mW0, mb0, mW1, mb1, mW2, mb2).reshape(E)
    acc = _edge_pass(q_all, k_all, g_all.reshape(N), edge_index[0],
                     edge_index[1], evT.reshape(3 * E), bias)
    out = _finalize(acc.reshape(NW * 4, NPAD))
    return out[:, :N].T


# speedup vs baseline: 1.2768x; 1.0775x over previous
import functools
import math

import jax
import jax.numpy as jnp
from jax import lax
from jax.experimental import pallas as pl
from jax.experimental.pallas import tpu as pltpu
from jax.experimental.pallas import tpu_sc as plsc

N = 10000
E = 320000
DIM = 128
NC, NS, L = 2, 16, 16
NW = NC * NS
EPW = E // NW
C = 80
NCHUNK = EPW // C
NPAD = 10240
ACC_LEN = NPAD * 4
INV_SQRT_D = 1.0 / math.sqrt(DIM)


def _node_proj_body(x_ref, wq_ref, bq_ref, wk_ref, bk_ref, wv_ref, bv_ref,
                    wf_ref, bf_ref, q_ref, k_ref, g_ref):
    x = x_ref[...]
    q_ref[...] = x @ wq_ref[...] + bq_ref[...]
    k_ref[...] = x @ wk_ref[...] + bk_ref[...]
    w_g = wv_ref[...] @ wf_ref[...]
    c_g = bv_ref[...] @ wf_ref[...] + bf_ref[...]
    g_ref[...] = x @ w_g + c_g


def _node_proj(x, W_Q, b_Q, W_K, b_K, W_V, b_V, wF_W, wF_b):
    return pl.pallas_call(
        _node_proj_body,
        out_shape=(
            jax.ShapeDtypeStruct((N, DIM), jnp.float32),
            jax.ShapeDtypeStruct((N, DIM), jnp.float32),
            jax.ShapeDtypeStruct((N, 1), jnp.float32),
        ),
    )(x, W_Q, b_Q.reshape(1, DIM), W_K, b_K.reshape(1, DIM),
      W_V, b_V.reshape(1, DIM), wF_W, wF_b.reshape(1, 1))


_EB = 6400


def _edge_mlp_body(ev_ref, w0_ref, b0_ref, w1_ref, b1_ref, w2_ref, b2_ref,
                   bias_ref):
    ev = ev_ref[...]
    ln = jnp.sqrt(jnp.sum(ev * ev, axis=0, keepdims=True))
    attr = jnp.concatenate([ev, ln], axis=0)
    h = w0_ref[...] @ attr + b0_ref[...]
    h = h * jax.nn.sigmoid(h)
    h = w1_ref[...] @ h + b1_ref[...]
    h = h * jax.nn.sigmoid(h)
    bias_ref[...] = (w2_ref[...] @ h + b2_ref[...]).reshape(1, 1, _EB)


def _edge_mlp(evT, mW0, mb0, mW1, mb1, mW2, mb2):
    nblk = E // _EB
    w_spec = lambda shp: pl.BlockSpec(shp, lambda e: (0, 0))
    return pl.pallas_call(
        _edge_mlp_body,
        grid=(nblk,),
        in_specs=[
            pl.BlockSpec((3, _EB), lambda e: (0, e)),
            w_spec((DIM, 4)), w_spec((DIM, 1)),
            w_spec((DIM, DIM)), w_spec((DIM, 1)),
            w_spec((1, DIM)), w_spec((1, 1)),
        ],
        out_specs=pl.BlockSpec((1, 1, _EB), lambda e: (e, 0, 0)),
        out_shape=jax.ShapeDtypeStruct((E // _EB, 1, _EB), jnp.float32),
    )(evT, mW0.T, mb0.reshape(DIM, 1), mW1.T, mb1.reshape(DIM, 1),
      mW2.T, mb2.reshape(1, 1))


def _splat(v):
    return jnp.full((L,), v, jnp.int32)


def _edge_pass_body(q_hbm, k_hbm, g_hbm, i_hbm, j_hbm, ev_hbm, bias_hbm,
                    out_hbm, i_v, j_v, g_v, qbuf0, kbuf0, evbuf0, qbuf1,
                    kbuf1, evbuf1, acc, sem_q, sem_k, sem_e):
    wid = lax.axis_index("s") * NC + lax.axis_index("c")
    base = wid * EPW

    zero = jnp.zeros((L,), jnp.float32)

    def zbody(t, carry):
        acc[pl.ds(t * L, L)] = zero
        return carry

    lax.fori_loop(0, ACC_LEN // L, zbody, 0, unroll=8)

    pltpu.sync_copy(i_hbm.at[pl.ds(base, EPW)], i_v)
    pltpu.sync_copy(j_hbm.at[pl.ds(base, EPW)], j_v)
    pltpu.sync_copy(g_hbm, g_v)

    e16 = lax.iota(jnp.int32, L)

    def copies(c, qb, kb, eb):
        off = c * C
        cps = [
            pltpu.make_async_copy(q_hbm.at[j_v.at[pl.ds(off, C)]], qb, sem_q),
            pltpu.make_async_copy(k_hbm.at[i_v.at[pl.ds(off, C)]], kb, sem_k),
        ]
        for p in range(3):
            cps.append(pltpu.make_async_copy(
                ev_hbm.at[pl.ds(p * E + base + off, C)],
                eb.at[pl.ds(p * C, C)], sem_e))
        cps.append(pltpu.make_async_copy(
            bias_hbm.at[pl.ds(base + off, C)],
            eb.at[pl.ds(3 * C, C)], sem_e))
        return cps

    def issue(c, qb, kb, eb):
        for cp in copies(c, qb, kb, eb):
            cp.start()

    def compute(c, qb, kb, eb):
        off = c * C
        for cp in copies(c, qb, kb, eb):
            cp.wait()
        for g in range(C // L):
            eg = g * L + e16

            def ebody(le, s16):
                e = g * L + le
                p = qb[e, pl.ds(0, L)] * kb[e, pl.ds(0, L)]
                for cc in range(1, DIM // L):
                    p = p + qb[e, pl.ds(cc * L, L)] * kb[e, pl.ds(cc * L, L)]
                tot = jnp.full((L,), plsc.cumsum(p)[L - 1])
                return jnp.where(e16 == le, tot, s16)

            s = lax.fori_loop(0, L, ebody, jnp.zeros((L,), jnp.float32),
                              unroll=4)
            s = s * INV_SQRT_D + eb[pl.ds(3 * C + g * L, L)]
            w = jnp.exp(s)
            iv = i_v[pl.ds(off + g * L, L)]
            gate = plsc.load_gather(g_v, [iv])
            ev0 = eb[pl.ds(g * L, L)]
            ev1 = eb[pl.ds(C + g * L, L)]
            ev2 = eb[pl.ds(2 * C + g * L, L)]
            wg = w * gate
            jv = j_v[pl.ds(off + g * L, L)]
            plsc.addupdate_scatter(acc, [jv], w)
            plsc.addupdate_scatter(acc, [jv + NPAD], wg * ev0)
            plsc.addupdate_scatter(acc, [jv + 2 * NPAD], wg * ev1)
            plsc.addupdate_scatter(acc, [jv + 3 * NPAD], wg * ev2)

    issue(0, qbuf0, kbuf0, evbuf0)

    def pair_body(t, carry):
        c = 2 * t
        issue(c + 1, qbuf1, kbuf1, evbuf1)
        compute(c, qbuf0, kbuf0, evbuf0)
        issue(c + 2, qbuf0, kbuf0, evbuf0)
        compute(c + 1, qbuf1, kbuf1, evbuf1)
        return carry

    lax.fori_loop(0, (NCHUNK - 1) // 2, pair_body, 0)
    compute(NCHUNK - 1, qbuf0, kbuf0, evbuf0)

    pltpu.sync_copy(acc, out_hbm.at[wid])


def _edge_pass(q_all, k_all, g_all, i_arr, j_arr, ev3_flat, bias_flat):
    mesh = plsc.VectorSubcoreMesh(core_axis_name="c", subcore_axis_name="s",
                                  num_cores=NC, num_subcores=NS)
    f = pl.kernel(
        _edge_pass_body,
        out_type=jax.ShapeDtypeStruct((NW, ACC_LEN), jnp.float32),
        mesh=mesh,
        scratch_types=[
            pltpu.VMEM((EPW,), jnp.int32),
            pltpu.VMEM((EPW,), jnp.int32),
            pltpu.VMEM((N,), jnp.float32),
            pltpu.VMEM((C, DIM), jnp.float32),
            pltpu.VMEM((C, DIM), jnp.float32),
            pltpu.VMEM((C * 4,), jnp.float32),
            pltpu.VMEM((C, DIM), jnp.float32),
            pltpu.VMEM((C, DIM), jnp.float32),
            pltpu.VMEM((C * 4,), jnp.float32),
            pltpu.VMEM((ACC_LEN,), jnp.float32),
            pltpu.SemaphoreType.DMA,
            pltpu.SemaphoreType.DMA,
            pltpu.SemaphoreType.DMA,
        ],
        compiler_params=pltpu.CompilerParams(needs_layout_passes=False),
    )
    return f(q_all, k_all, g_all, i_arr, j_arr, ev3_flat, bias_flat)


def _finalize_body(acc_ref, out_ref):
    a = acc_ref[...].reshape(NW, 4, NPAD)
    a = jnp.sum(a, axis=0)
    out_ref[...] = a[1:4] / (a[0:1] + 1e-16)


def _finalize(acc_all):
    return pl.pallas_call(
        _finalize_body,
        out_shape=jax.ShapeDtypeStruct((3, NPAD), jnp.float32),
    )(acc_all)


def kernel(x, edge_index, edge_vec, W_Q, b_Q, W_K, b_K, W_V, b_V,
           mW0, mb0, mW1, mb1, mW2, mb2, wF_W, wF_b):
    q_all, k_all, g_all = _node_proj(x, W_Q, b_Q, W_K, b_K, W_V, b_V,
                                     wF_W, wF_b)
    evT = edge_vec.T
    bias = _edge_mlp(evT, mW0, mb0, mW1, mb1, mW2, mb2).reshape(E)
    acc = _edge_pass(q_all, k_all, g_all.reshape(N), edge_index[0],
                     edge_index[1], evT.reshape(3 * E), bias)
    out = _finalize(acc.reshape(NW * 4, NPAD))
    return out[:, :N].T
